# baseline (speedup 1.0000x reference)
import functools

import jax
import jax.numpy as jnp
import numpy as np
from jax import lax
from jax.experimental import pallas as pl
from jax.experimental.pallas import tpu as pltpu
from jax.experimental.pallas import tpu_sc as plsc

N = 10000
E = 320000
D = 128
H = 128
OUT = 64
G = 64
NUM_LAYERS = 3

NC = 2
NS = 16
NW = NC * NS
CHUNK = 64
NBUF = 4
ECH = E // CHUNK
CPT = 156
NSTG = 3
SPC = CPT // NSTG
XTRA = ECH - NW * CPT
NPAD = 10240
RPT = NPAD // NS


def _sc_agg_body(h_hbm, src_hbm, dst_hbm, out_hbm, agg_sh, src_v, dst_v,
                 *scr):
    bufs = scr[:NBUF]
    gsems = scr[NBUF:2 * NBUF]
    ssems = scr[2 * NBUF:3 * NBUF]
    c = lax.axis_index("c")
    s = lax.axis_index("s")
    wid = c * NS + s

    @pl.loop(0, CHUNK)
    def _zr(i):
        @pl.loop(0, H, step=16)
        def _zc(j):
            bufs[0][i, pl.ds(j, 16)] = jnp.zeros((16,), jnp.float32)

    row0 = s * RPT

    @pl.loop(0, RPT, step=CHUNK)
    def _zs(r):
        pltpu.sync_copy(bufs[0], agg_sh.at[pl.ds(row0 + r, CHUNK)])

    plsc.subcore_barrier()

    def _start_gather(j, b):
        pltpu.async_copy(h_hbm.at[src_v.at[pl.ds(j * CHUNK, CHUNK)]],
                         bufs[b], gsems[b])

    def _start_scatter(j, b):
        pltpu.async_copy(bufs[b],
                         agg_sh.at[dst_v.at[pl.ds(j * CHUNK, CHUNK)]],
                         ssems[b], add=True)

    def _wait(b, sems):
        pltpu.make_async_copy(h_hbm.at[pl.ds(0, CHUNK)], bufs[b],
                              sems[b]).wait()

    for st in range(NSTG):
        eb = (wid * CPT + st * SPC) * CHUNK
        pltpu.sync_copy(src_hbm.at[pl.ds(eb, SPC * CHUNK)], src_v)
        pltpu.sync_copy(dst_hbm.at[pl.ds(eb, SPC * CHUNK)], dst_v)
        for b in range(NBUF):
            _start_gather(b, b)

        @pl.loop(0, SPC - NBUF, step=NBUF)
        def _mn(j):
            _wait(0, gsems)
            _start_scatter(j, 0)
            _wait(1, gsems)
            _start_scatter(j + 1, 1)
            _wait(0, ssems)
            _start_gather(j + NBUF, 0)
            _wait(2, gsems)
            _start_scatter(j + 2, 2)
            _wait(1, ssems)
            _start_gather(j + NBUF + 1, 1)
            _wait(3, gsems)
            _start_scatter(j + 3, 3)
            _wait(2, ssems)
            _start_gather(j + NBUF + 2, 2)
            _wait(3, ssems)
            _start_gather(j + NBUF + 3, 3)

        for b in range(NBUF):
            _wait(b, gsems)
            _start_scatter(SPC - NBUF + b, b)
        for b in range(NBUF):
            _wait(b, ssems)

    @pl.when(wid < XTRA)
    def _tail():
        teb = (NW * CPT + wid) * CHUNK
        pltpu.sync_copy(src_hbm.at[pl.ds(teb, CHUNK)],
                        src_v.at[pl.ds(0, CHUNK)])
        pltpu.sync_copy(dst_hbm.at[pl.ds(teb, CHUNK)],
                        dst_v.at[pl.ds(0, CHUNK)])
        pltpu.sync_copy(h_hbm.at[src_v.at[pl.ds(0, CHUNK)]], bufs[0])
        pltpu.sync_copy(bufs[0], agg_sh.at[dst_v.at[pl.ds(0, CHUNK)]],
                        add=True)

    plsc.subcore_barrier()
    pltpu.sync_copy(agg_sh.at[pl.ds(row0, RPT)],
                    out_hbm.at[c, pl.ds(row0, RPT)])


@jax.jit
def _sc_agg(h, src3, dst3):
    mesh = plsc.VectorSubcoreMesh(core_axis_name="c", subcore_axis_name="s")
    f = pl.kernel(
        _sc_agg_body,
        mesh=mesh,
        out_type=jax.ShapeDtypeStruct((NC, NPAD, H), jnp.float32),
        scratch_types=[
            pltpu.VMEM_SHARED((NPAD, H), jnp.float32),
            pltpu.VMEM((SPC * CHUNK,), jnp.int32),
            pltpu.VMEM((SPC * CHUNK,), jnp.int32),
        ] + [pltpu.VMEM((CHUNK, H), jnp.float32) for _ in range(NBUF)]
          + [pltpu.SemaphoreType.DMA for _ in range(2 * NBUF)],
    )
    return f(h, src3, dst3)


def _tc_layer_body(h_ref, aggs_ref, eps_ref, W1_ref, b1_ref, W2_ref, b2_ref,
                   g_ref, be_ref, out_ref):
    h = h_ref[...]
    agg = aggs_ref[0, :N, :] + aggs_ref[1, :N, :]
    z = (1.0 + eps_ref[...]) * h + agg
    a = jnp.maximum(
        jnp.dot(z, W1_ref[...], preferred_element_type=jnp.float32)
        + b1_ref[...], 0.0)
    b = jnp.maximum(
        jnp.dot(a, W2_ref[...], preferred_element_type=jnp.float32)
        + b2_ref[...], 0.0)
    mean = jnp.mean(b, axis=0)
    var = jnp.mean(b * b, axis=0) - mean * mean
    out_ref[...] = (b - mean) * lax.rsqrt(var + 1e-5) * g_ref[...] + be_ref[...]


@jax.jit
def _tc_layer(h, aggs, epsb, W1, b1, W2, b2, g, be):
    return pl.pallas_call(
        _tc_layer_body,
        out_shape=jax.ShapeDtypeStruct((N, H), jnp.float32),
    )(h, aggs, epsb, W1, b1, W2, b2, g, be)


def _tc_final_body(h_ref, aggs_ref, eps_ref, W1_ref, b1_ref, W2_ref, b2_ref,
                   g_ref, be_ref, batch_ref, l1W_ref, l1b_ref, l2W_ref,
                   l2b_ref, out_ref):
    h = h_ref[...]
    agg = aggs_ref[0, :N, :] + aggs_ref[1, :N, :]
    z = (1.0 + eps_ref[...]) * h + agg
    a = jnp.maximum(
        jnp.dot(z, W1_ref[...], preferred_element_type=jnp.float32)
        + b1_ref[...], 0.0)
    b = jnp.maximum(
        jnp.dot(a, W2_ref[...], preferred_element_type=jnp.float32)
        + b2_ref[...], 0.0)
    mean = jnp.mean(b, axis=0)
    var = jnp.mean(b * b, axis=0) - mean * mean
    hn = (b - mean) * lax.rsqrt(var + 1e-5) * g_ref[...] + be_ref[...]
    bids = batch_ref[0, :]
    onehot = (lax.broadcasted_iota(jnp.int32, (G, N), 0)
              == bids[None, :]).astype(jnp.float32)
    sums = jnp.dot(onehot, hn, preferred_element_type=jnp.float32)
    cnt = jnp.sum(onehot, axis=1)
    pooled = sums / jnp.maximum(cnt, 1.0)[:, None]
    t = jnp.maximum(
        jnp.dot(pooled, l1W_ref[...], preferred_element_type=jnp.float32)
        + l1b_ref[...], 0.0)
    o = jnp.dot(t, l2W_ref[...], preferred_element_type=jnp.float32) \
        + l2b_ref[...]
    m = jnp.max(o, axis=1, keepdims=True)
    lse = jnp.log(jnp.sum(jnp.exp(o - m), axis=1, keepdims=True)) + m
    out_ref[...] = o - lse


@jax.jit
def _tc_final(h, aggs, epsb, W1, b1, W2, b2, g, be, batch2, l1W, l1b, l2W,
              l2b):
    return pl.pallas_call(
        _tc_final_body,
        out_shape=jax.ShapeDtypeStruct((G, OUT), jnp.float32),
    )(h, aggs, epsb, W1, b1, W2, b2, g, be, batch2, l1W, l1b, l2W, l2b)


def kernel(x, edge_index, batch,
           W1_0, b1_0, W2_0, b2_0, g_0, be_0, eps_0,
           W1_1, b1_1, W2_1, b2_1, g_1, be_1, eps_1,
           W1_2, b1_2, W2_2, b2_2, g_2, be_2, eps_2,
           lin1_W, lin1_b, lin2_W, lin2_b):
    srcR = edge_index[0]
    dstR = edge_index[1]
    batch2 = batch.reshape(1, N)

    params = [
        (W1_0, b1_0, W2_0, b2_0, g_0, be_0, eps_0),
        (W1_1, b1_1, W2_1, b2_1, g_1, be_1, eps_1),
        (W1_2, b1_2, W2_2, b2_2, g_2, be_2, eps_2),
    ]
    h = x
    for l in range(NUM_LAYERS):
        W1, b1, W2, b2, g, be, eps = params[l]
        aggs = _sc_agg(h, srcR, dstR)
        epsb = jnp.broadcast_to(eps.reshape(1, 1), (1, H))
        b1r, b2r = b1.reshape(1, H), b2.reshape(1, H)
        gr, ber = g.reshape(1, H), be.reshape(1, H)
        if l < NUM_LAYERS - 1:
            h = _tc_layer(h, aggs, epsb, W1, b1r, W2, b2r, gr, ber)
        else:
            out = _tc_final(h, aggs, epsb, W1, b1r, W2, b2r, gr, ber,
                            batch2, lin1_W, lin1_b.reshape(1, H), lin2_W,
                            lin2_b.reshape(1, OUT))
    return out


# speedup vs baseline: 1.0450x; 1.0450x over previous
import functools

import jax
import jax.numpy as jnp
import numpy as np
from jax import lax
from jax.experimental import pallas as pl
from jax.experimental.pallas import tpu as pltpu
from jax.experimental.pallas import tpu_sc as plsc

N = 10000
E = 320000
D = 128
H = 128
OUT = 64
G = 64
NUM_LAYERS = 3

NC = 2
NS = 16
NW = NC * NS
CHUNK = 64
NBUF = 4
ECH = E // CHUNK
CPT = 156
NSTG = 3
SPC = CPT // NSTG
XTRA = ECH - NW * CPT
NPAD = 10240
RPT = NPAD // NS


def _sc_agg_body(h_hbm, ei_hbm, out_hbm, agg_sh, sd_v, *scr):
    bufs = scr[:NBUF]
    gsems = scr[NBUF:2 * NBUF]
    ssems = scr[2 * NBUF:3 * NBUF]
    c = lax.axis_index("c")
    s = lax.axis_index("s")
    wid = c * NS + s

    @pl.loop(0, CHUNK)
    def _zr(i):
        @pl.loop(0, H, step=16)
        def _zc(j):
            bufs[0][i, pl.ds(j, 16)] = jnp.zeros((16,), jnp.float32)

    row0 = s * RPT

    @pl.loop(0, RPT, step=CHUNK)
    def _zs(r):
        pltpu.sync_copy(bufs[0], agg_sh.at[pl.ds(row0 + r, CHUNK)])

    plsc.subcore_barrier()

    def _start_gather(j, b):
        pltpu.async_copy(h_hbm.at[sd_v.at[0, pl.ds(j * CHUNK, CHUNK)]],
                         bufs[b], gsems[b])

    def _start_scatter(j, b):
        pltpu.async_copy(bufs[b],
                         agg_sh.at[sd_v.at[1, pl.ds(j * CHUNK, CHUNK)]],
                         ssems[b], add=True)

    def _wait(b, sems):
        pltpu.make_async_copy(h_hbm.at[pl.ds(0, CHUNK)], bufs[b],
                              sems[b]).wait()

    for st in range(NSTG):
        eb = (wid * CPT + st * SPC) * CHUNK
        pltpu.sync_copy(ei_hbm.at[:, pl.ds(eb, SPC * CHUNK)], sd_v)
        for b in range(NBUF):
            _start_gather(b, b)

        @pl.loop(0, SPC - NBUF, step=NBUF)
        def _mn(j):
            _wait(0, gsems)
            _start_scatter(j, 0)
            _wait(1, gsems)
            _start_scatter(j + 1, 1)
            _wait(0, ssems)
            _start_gather(j + NBUF, 0)
            _wait(2, gsems)
            _start_scatter(j + 2, 2)
            _wait(1, ssems)
            _start_gather(j + NBUF + 1, 1)
            _wait(3, gsems)
            _start_scatter(j + 3, 3)
            _wait(2, ssems)
            _start_gather(j + NBUF + 2, 2)
            _wait(3, ssems)
            _start_gather(j + NBUF + 3, 3)

        for b in range(NBUF):
            _wait(b, gsems)
            _start_scatter(SPC - NBUF + b, b)
        for b in range(NBUF):
            _wait(b, ssems)

    @pl.when(wid < XTRA // 2)
    def _tail():
        teb = NW * CPT * CHUNK + wid * (2 * CHUNK)
        pltpu.sync_copy(ei_hbm.at[:, pl.ds(teb, 2 * CHUNK)],
                        sd_v.at[:, pl.ds(0, 2 * CHUNK)])
        for q in range(2):
            pltpu.sync_copy(
                h_hbm.at[sd_v.at[0, pl.ds(q * CHUNK, CHUNK)]], bufs[q])
            pltpu.sync_copy(
                bufs[q], agg_sh.at[sd_v.at[1, pl.ds(q * CHUNK, CHUNK)]],
                add=True)

    plsc.subcore_barrier()
    pltpu.sync_copy(agg_sh.at[pl.ds(row0, RPT)],
                    out_hbm.at[c, pl.ds(row0, RPT)])


@jax.jit
def _sc_agg(h, ei):
    mesh = plsc.VectorSubcoreMesh(core_axis_name="c", subcore_axis_name="s")
    f = pl.kernel(
        _sc_agg_body,
        mesh=mesh,
        out_type=jax.ShapeDtypeStruct((NC, NPAD, H), jnp.float32),
        scratch_types=[
            pltpu.VMEM_SHARED((NPAD, H), jnp.float32),
            pltpu.VMEM((2, SPC * CHUNK), jnp.int32),
        ] + [pltpu.VMEM((CHUNK, H), jnp.float32) for _ in range(NBUF)]
          + [pltpu.SemaphoreType.DMA for _ in range(2 * NBUF)],
    )
    return f(h, ei)


def _tc_layer_body(h_ref, aggs_ref, eps_ref, W1_ref, b1_ref, W2_ref, b2_ref,
                   g_ref, be_ref, out_ref):
    h = h_ref[...]
    agg = aggs_ref[0, :N, :] + aggs_ref[1, :N, :]
    z = (1.0 + eps_ref[...]) * h + agg
    a = jnp.maximum(
        jnp.dot(z, W1_ref[...], preferred_element_type=jnp.float32)
        + b1_ref[...], 0.0)
    b = jnp.maximum(
        jnp.dot(a, W2_ref[...], preferred_element_type=jnp.float32)
        + b2_ref[...], 0.0)
    mean = jnp.mean(b, axis=0)
    var = jnp.mean(b * b, axis=0) - mean * mean
    out_ref[...] = (b - mean) * lax.rsqrt(var + 1e-5) * g_ref[...] + be_ref[...]


@jax.jit
def _tc_layer(h, aggs, epsb, W1, b1, W2, b2, g, be):
    return pl.pallas_call(
        _tc_layer_body,
        out_shape=jax.ShapeDtypeStruct((N, H), jnp.float32),
    )(h, aggs, epsb, W1, b1, W2, b2, g, be)


def _tc_final_body(h_ref, aggs_ref, eps_ref, W1_ref, b1_ref, W2_ref, b2_ref,
                   g_ref, be_ref, batch_ref, l1W_ref, l1b_ref, l2W_ref,
                   l2b_ref, out_ref):
    h = h_ref[...]
    agg = aggs_ref[0, :N, :] + aggs_ref[1, :N, :]
    z = (1.0 + eps_ref[...]) * h + agg
    a = jnp.maximum(
        jnp.dot(z, W1_ref[...], preferred_element_type=jnp.float32)
        + b1_ref[...], 0.0)
    b = jnp.maximum(
        jnp.dot(a, W2_ref[...], preferred_element_type=jnp.float32)
        + b2_ref[...], 0.0)
    mean = jnp.mean(b, axis=0)
    var = jnp.mean(b * b, axis=0) - mean * mean
    hn = (b - mean) * lax.rsqrt(var + 1e-5) * g_ref[...] + be_ref[...]
    bids = batch_ref[0, :]
    onehot = (lax.broadcasted_iota(jnp.int32, (G, N), 0)
              == bids[None, :]).astype(jnp.float32)
    sums = jnp.dot(onehot, hn, preferred_element_type=jnp.float32)
    cnt = jnp.sum(onehot, axis=1)
    pooled = sums / jnp.maximum(cnt, 1.0)[:, None]
    t = jnp.maximum(
        jnp.dot(pooled, l1W_ref[...], preferred_element_type=jnp.float32)
        + l1b_ref[...], 0.0)
    o = jnp.dot(t, l2W_ref[...], preferred_element_type=jnp.float32) \
        + l2b_ref[...]
    m = jnp.max(o, axis=1, keepdims=True)
    lse = jnp.log(jnp.sum(jnp.exp(o - m), axis=1, keepdims=True)) + m
    out_ref[...] = o - lse


@jax.jit
def _tc_final(h, aggs, epsb, W1, b1, W2, b2, g, be, batch2, l1W, l1b, l2W,
              l2b):
    return pl.pallas_call(
        _tc_final_body,
        out_shape=jax.ShapeDtypeStruct((G, OUT), jnp.float32),
    )(h, aggs, epsb, W1, b1, W2, b2, g, be, batch2, l1W, l1b, l2W, l2b)


def kernel(x, edge_index, batch,
           W1_0, b1_0, W2_0, b2_0, g_0, be_0, eps_0,
           W1_1, b1_1, W2_1, b2_1, g_1, be_1, eps_1,
           W1_2, b1_2, W2_2, b2_2, g_2, be_2, eps_2,
           lin1_W, lin1_b, lin2_W, lin2_b):
    batch2 = batch.reshape(1, N)

    params = [
        (W1_0, b1_0, W2_0, b2_0, g_0, be_0, eps_0),
        (W1_1, b1_1, W2_1, b2_1, g_1, be_1, eps_1),
        (W1_2, b1_2, W2_2, b2_2, g_2, be_2, eps_2),
    ]
    h = x
    for l in range(NUM_LAYERS):
        W1, b1, W2, b2, g, be, eps = params[l]
        aggs = _sc_agg(h, edge_index)
        epsb = jnp.broadcast_to(eps.reshape(1, 1), (1, H))
        b1r, b2r = b1.reshape(1, H), b2.reshape(1, H)
        gr, ber = g.reshape(1, H), be.reshape(1, H)
        if l < NUM_LAYERS - 1:
            h = _tc_layer(h, aggs, epsb, W1, b1r, W2, b2r, gr, ber)
        else:
            out = _tc_final(h, aggs, epsb, W1, b1r, W2, b2r, gr, ber,
                            batch2, lin1_W, lin1_b.reshape(1, H), lin2_W,
                            lin2_b.reshape(1, OUT))
    return out


# speedup vs baseline: 1.0549x; 1.0095x over previous
import functools

import jax
import jax.numpy as jnp
import numpy as np
from jax import lax
from jax.experimental import pallas as pl
from jax.experimental.pallas import tpu as pltpu
from jax.experimental.pallas import tpu_sc as plsc

N = 10000
E = 320000
D = 128
H = 128
OUT = 64
G = 64
NUM_LAYERS = 3

NC = 2
NS = 16
NW = NC * NS
CHUNK = 64
NBUF = 4
ECH = E // CHUNK
CPT = 156
NSTG = 3
SPC = CPT // NSTG
XTRA = ECH - NW * CPT
NPAD = 10240
RPT = NPAD // NS


def _sc_agg_body(h_hbm, ei_hbm, out_hbm, agg_sh, sd_v, *scr):
    bufs = scr[:NBUF]
    gsems = scr[NBUF:2 * NBUF]
    ssems = scr[2 * NBUF:3 * NBUF]
    c = lax.axis_index("c")
    s = lax.axis_index("s")
    wid = c * NS + s

    pltpu.async_copy(ei_hbm.at[:, pl.ds(wid * CPT * CHUNK, SPC * CHUNK)],
                     sd_v, gsems[0])

    @pl.loop(0, CHUNK)
    def _zr(i):
        @pl.loop(0, H, step=16)
        def _zc(j):
            bufs[0][i, pl.ds(j, 16)] = jnp.zeros((16,), jnp.float32)

    row0 = s * RPT
    nz = RPT // CHUNK
    for r in range(nz):
        pltpu.async_copy(bufs[0], agg_sh.at[pl.ds(row0 + r * CHUNK, CHUNK)],
                         ssems[r % NBUF])
    for b in range(NBUF):
        for _ in range((nz - b + NBUF - 1) // NBUF):
            pltpu.make_async_copy(h_hbm.at[pl.ds(0, CHUNK)], bufs[b],
                                  ssems[b]).wait()
    pltpu.make_async_copy(ei_hbm.at[:, pl.ds(0, SPC * CHUNK)], sd_v,
                          gsems[0]).wait()

    plsc.subcore_barrier()

    def _start_gather(j, b):
        pltpu.async_copy(h_hbm.at[sd_v.at[0, pl.ds(j * CHUNK, CHUNK)]],
                         bufs[b], gsems[b])

    def _start_scatter(j, b):
        pltpu.async_copy(bufs[b],
                         agg_sh.at[sd_v.at[1, pl.ds(j * CHUNK, CHUNK)]],
                         ssems[b], add=True)

    def _wait(b, sems):
        pltpu.make_async_copy(h_hbm.at[pl.ds(0, CHUNK)], bufs[b],
                              sems[b]).wait()

    for st in range(NSTG):
        if st > 0:
            eb = (wid * CPT + st * SPC) * CHUNK
            pltpu.sync_copy(ei_hbm.at[:, pl.ds(eb, SPC * CHUNK)], sd_v)
        for b in range(NBUF):
            _start_gather(b, b)

        @pl.loop(0, SPC - NBUF, step=NBUF)
        def _mn(j):
            _wait(0, gsems)
            _start_scatter(j, 0)
            _wait(1, gsems)
            _start_scatter(j + 1, 1)
            _wait(0, ssems)
            _start_gather(j + NBUF, 0)
            _wait(2, gsems)
            _start_scatter(j + 2, 2)
            _wait(1, ssems)
            _start_gather(j + NBUF + 1, 1)
            _wait(3, gsems)
            _start_scatter(j + 3, 3)
            _wait(2, ssems)
            _start_gather(j + NBUF + 2, 2)
            _wait(3, ssems)
            _start_gather(j + NBUF + 3, 3)

        for b in range(NBUF):
            _wait(b, gsems)
            _start_scatter(SPC - NBUF + b, b)
        for b in range(NBUF):
            _wait(b, ssems)

    @pl.when(wid < XTRA // 2)
    def _tail():
        teb = NW * CPT * CHUNK + wid * (2 * CHUNK)
        pltpu.sync_copy(ei_hbm.at[:, pl.ds(teb, 2 * CHUNK)],
                        sd_v.at[:, pl.ds(0, 2 * CHUNK)])
        for q in range(2):
            pltpu.sync_copy(
                h_hbm.at[sd_v.at[0, pl.ds(q * CHUNK, CHUNK)]], bufs[q])
            pltpu.sync_copy(
                bufs[q], agg_sh.at[sd_v.at[1, pl.ds(q * CHUNK, CHUNK)]],
                add=True)

    plsc.subcore_barrier()
    pltpu.sync_copy(agg_sh.at[pl.ds(row0, RPT)],
                    out_hbm.at[c, pl.ds(row0, RPT)])


@jax.jit
def _sc_agg(h, ei):
    mesh = plsc.VectorSubcoreMesh(core_axis_name="c", subcore_axis_name="s")
    f = pl.kernel(
        _sc_agg_body,
        mesh=mesh,
        out_type=jax.ShapeDtypeStruct((NC, NPAD, H), jnp.float32),
        scratch_types=[
            pltpu.VMEM_SHARED((NPAD, H), jnp.float32),
            pltpu.VMEM((2, SPC * CHUNK), jnp.int32),
        ] + [pltpu.VMEM((CHUNK, H), jnp.float32) for _ in range(NBUF)]
          + [pltpu.SemaphoreType.DMA for _ in range(2 * NBUF)],
    )
    return f(h, ei)


def _tc_layer_body(h_ref, aggs_ref, eps_ref, W1_ref, b1_ref, W2_ref, b2_ref,
                   g_ref, be_ref, out_ref):
    h = h_ref[...]
    agg = aggs_ref[0, :N, :] + aggs_ref[1, :N, :]
    z = (1.0 + eps_ref[...]) * h + agg
    a = jnp.maximum(
        jnp.dot(z, W1_ref[...], preferred_element_type=jnp.float32)
        + b1_ref[...], 0.0)
    b = jnp.maximum(
        jnp.dot(a, W2_ref[...], preferred_element_type=jnp.float32)
        + b2_ref[...], 0.0)
    mean = jnp.mean(b, axis=0)
    var = jnp.mean(b * b, axis=0) - mean * mean
    out_ref[...] = (b - mean) * lax.rsqrt(var + 1e-5) * g_ref[...] + be_ref[...]


@jax.jit
def _tc_layer(h, aggs, epsb, W1, b1, W2, b2, g, be):
    return pl.pallas_call(
        _tc_layer_body,
        out_shape=jax.ShapeDtypeStruct((N, H), jnp.float32),
    )(h, aggs, epsb, W1, b1, W2, b2, g, be)


def _tc_final_body(h_ref, aggs_ref, eps_ref, W1_ref, b1_ref, W2_ref, b2_ref,
                   g_ref, be_ref, batch_ref, l1W_ref, l1b_ref, l2W_ref,
                   l2b_ref, out_ref):
    h = h_ref[...]
    agg = aggs_ref[0, :N, :] + aggs_ref[1, :N, :]
    z = (1.0 + eps_ref[...]) * h + agg
    a = jnp.maximum(
        jnp.dot(z, W1_ref[...], preferred_element_type=jnp.float32)
        + b1_ref[...], 0.0)
    b = jnp.maximum(
        jnp.dot(a, W2_ref[...], preferred_element_type=jnp.float32)
        + b2_ref[...], 0.0)
    mean = jnp.mean(b, axis=0)
    var = jnp.mean(b * b, axis=0) - mean * mean
    hn = (b - mean) * lax.rsqrt(var + 1e-5) * g_ref[...] + be_ref[...]
    bids = batch_ref[0, :]
    onehot = (lax.broadcasted_iota(jnp.int32, (G, N), 0)
              == bids[None, :]).astype(jnp.float32)
    sums = jnp.dot(onehot, hn, preferred_element_type=jnp.float32)
    cnt = jnp.sum(onehot, axis=1)
    pooled = sums / jnp.maximum(cnt, 1.0)[:, None]
    t = jnp.maximum(
        jnp.dot(pooled, l1W_ref[...], preferred_element_type=jnp.float32)
        + l1b_ref[...], 0.0)
    o = jnp.dot(t, l2W_ref[...], preferred_element_type=jnp.float32) \
        + l2b_ref[...]
    m = jnp.max(o, axis=1, keepdims=True)
    lse = jnp.log(jnp.sum(jnp.exp(o - m), axis=1, keepdims=True)) + m
    out_ref[...] = o - lse


@jax.jit
def _tc_final(h, aggs, epsb, W1, b1, W2, b2, g, be, batch2, l1W, l1b, l2W,
              l2b):
    return pl.pallas_call(
        _tc_final_body,
        out_shape=jax.ShapeDtypeStruct((G, OUT), jnp.float32),
    )(h, aggs, epsb, W1, b1, W2, b2, g, be, batch2, l1W, l1b, l2W, l2b)


def kernel(x, edge_index, batch,
           W1_0, b1_0, W2_0, b2_0, g_0, be_0, eps_0,
           W1_1, b1_1, W2_1, b2_1, g_1, be_1, eps_1,
           W1_2, b1_2, W2_2, b2_2, g_2, be_2, eps_2,
           lin1_W, lin1_b, lin2_W, lin2_b):
    batch2 = batch.reshape(1, N)

    params = [
        (W1_0, b1_0, W2_0, b2_0, g_0, be_0, eps_0),
        (W1_1, b1_1, W2_1, b2_1, g_1, be_1, eps_1),
        (W1_2, b1_2, W2_2, b2_2, g_2, be_2, eps_2),
    ]
    h = x
    for l in range(NUM_LAYERS):
        W1, b1, W2, b2, g, be, eps = params[l]
        aggs = _sc_agg(h, edge_index)
        epsb = jnp.broadcast_to(eps.reshape(1, 1), (1, H))
        b1r, b2r = b1.reshape(1, H), b2.reshape(1, H)
        gr, ber = g.reshape(1, H), be.reshape(1, H)
        if l < NUM_LAYERS - 1:
            h = _tc_layer(h, aggs, epsb, W1, b1r, W2, b2r, gr, ber)
        else:
            out = _tc_final(h, aggs, epsb, W1, b1r, W2, b2r, gr, ber,
                            batch2, lin1_W, lin1_b.reshape(1, H), lin2_W,
                            lin2_b.reshape(1, OUT))
    return out


# speedup vs baseline: 1.0556x; 1.0007x over previous
import jax
import jax.numpy as jnp
from jax import lax
from jax.experimental import pallas as pl
from jax.experimental.pallas import tpu as pltpu
from jax.experimental.pallas import tpu_sc as plsc

N = 10000
E = 320000
D = 128
H = 128
OUT = 64
G = 64
NUM_LAYERS = 3

NC = 2
NS = 16
NW = NC * NS
CHUNK = 64
NBUF = 4
ECH = E // CHUNK
CPT = 156
NSTG = 3
SPC = CPT // NSTG
XTRA = ECH - NW * CPT
NPAD = 10240
RPT = NPAD // NS


def _sc_agg_body(h_hbm, ei_hbm, out_hbm, agg_sh, sd_v, *scr):
    bufs = scr[:NBUF]
    gsems = scr[NBUF:2 * NBUF]
    ssems = scr[2 * NBUF:3 * NBUF]
    c = lax.axis_index("c")
    s = lax.axis_index("s")
    wid = c * NS + s

    pltpu.async_copy(ei_hbm.at[:, pl.ds(wid * CPT * CHUNK, SPC * CHUNK)],
                     sd_v, gsems[0])

    @pl.loop(0, CHUNK)
    def _zr(i):
        @pl.loop(0, H, step=16)
        def _zc(j):
            bufs[0][i, pl.ds(j, 16)] = jnp.zeros((16,), jnp.float32)

    row0 = s * RPT
    nz = RPT // CHUNK
    for r in range(nz):
        pltpu.async_copy(bufs[0], agg_sh.at[pl.ds(row0 + r * CHUNK, CHUNK)],
                         ssems[r % NBUF])
    for b in range(NBUF):
        for _ in range((nz - b + NBUF - 1) // NBUF):
            pltpu.make_async_copy(h_hbm.at[pl.ds(0, CHUNK)], bufs[b],
                                  ssems[b]).wait()
    pltpu.make_async_copy(ei_hbm.at[:, pl.ds(0, SPC * CHUNK)], sd_v,
                          gsems[0]).wait()

    plsc.subcore_barrier()

    def _start_gather(j, b):
        pltpu.async_copy(h_hbm.at[sd_v.at[0, pl.ds(j * CHUNK, CHUNK)]],
                         bufs[b], gsems[b])

    def _start_scatter(j, b):
        pltpu.async_copy(bufs[b],
                         agg_sh.at[sd_v.at[1, pl.ds(j * CHUNK, CHUNK)]],
                         ssems[b], add=True)

    def _wait(b, sems):
        pltpu.make_async_copy(h_hbm.at[pl.ds(0, CHUNK)], bufs[b],
                              sems[b]).wait()

    for st in range(NSTG):
        if st > 0:
            eb = (wid * CPT + st * SPC) * CHUNK
            pltpu.sync_copy(ei_hbm.at[:, pl.ds(eb, SPC * CHUNK)], sd_v)
        for b in range(NBUF):
            _start_gather(b, b)

        @pl.loop(0, SPC - NBUF, step=NBUF)
        def _mn(j):
            _wait(0, gsems)
            _start_scatter(j, 0)
            _wait(1, gsems)
            _start_scatter(j + 1, 1)
            _wait(0, ssems)
            _start_gather(j + NBUF, 0)
            _wait(2, gsems)
            _start_scatter(j + 2, 2)
            _wait(1, ssems)
            _start_gather(j + NBUF + 1, 1)
            _wait(3, gsems)
            _start_scatter(j + 3, 3)
            _wait(2, ssems)
            _start_gather(j + NBUF + 2, 2)
            _wait(3, ssems)
            _start_gather(j + NBUF + 3, 3)

        for b in range(NBUF):
            _wait(b, gsems)
            _start_scatter(SPC - NBUF + b, b)
        for b in range(NBUF):
            _wait(b, ssems)

    @pl.when(wid < XTRA // 2)
    def _tail():
        teb = NW * CPT * CHUNK + wid * (2 * CHUNK)
        pltpu.sync_copy(ei_hbm.at[:, pl.ds(teb, 2 * CHUNK)],
                        sd_v.at[:, pl.ds(0, 2 * CHUNK)])
        for q in range(2):
            pltpu.sync_copy(
                h_hbm.at[sd_v.at[0, pl.ds(q * CHUNK, CHUNK)]], bufs[q])
            pltpu.sync_copy(
                bufs[q], agg_sh.at[sd_v.at[1, pl.ds(q * CHUNK, CHUNK)]],
                add=True)

    plsc.subcore_barrier()
    pltpu.sync_copy(agg_sh.at[pl.ds(row0, RPT)],
                    out_hbm.at[c, pl.ds(row0, RPT)])


@jax.jit
def _sc_agg(h, ei):
    mesh = plsc.VectorSubcoreMesh(core_axis_name="c", subcore_axis_name="s")
    f = pl.kernel(
        _sc_agg_body,
        mesh=mesh,
        out_type=jax.ShapeDtypeStruct((NC, NPAD, H), jnp.float32),
        scratch_types=[
            pltpu.VMEM_SHARED((NPAD, H), jnp.float32),
            pltpu.VMEM((2, SPC * CHUNK), jnp.int32),
        ] + [pltpu.VMEM((CHUNK, H), jnp.float32) for _ in range(NBUF)]
          + [pltpu.SemaphoreType.DMA for _ in range(2 * NBUF)],
    )
    return f(h, ei)


def _tc_layer_body(h_ref, aggs_ref, eps_ref, W1_ref, b1_ref, W2_ref, b2_ref,
                   g_ref, be_ref, out_ref):
    h = h_ref[...]
    agg = aggs_ref[0, :N, :] + aggs_ref[1, :N, :]
    z = (1.0 + eps_ref[...]) * h + agg
    a = jnp.maximum(
        jnp.dot(z, W1_ref[...], preferred_element_type=jnp.float32)
        + b1_ref[...], 0.0)
    b = jnp.maximum(
        jnp.dot(a, W2_ref[...], preferred_element_type=jnp.float32)
        + b2_ref[...], 0.0)
    mean = jnp.mean(b, axis=0)
    var = jnp.mean(b * b, axis=0) - mean * mean
    out_ref[...] = (b - mean) * lax.rsqrt(var + 1e-5) * g_ref[...] + be_ref[...]


@jax.jit
def _tc_layer(h, aggs, epsb, W1, b1, W2, b2, g, be):
    return pl.pallas_call(
        _tc_layer_body,
        out_shape=jax.ShapeDtypeStruct((N, H), jnp.float32),
    )(h, aggs, epsb, W1, b1, W2, b2, g, be)


def _tc_final_body(h_ref, aggs_ref, eps_ref, W1_ref, b1_ref, W2_ref, b2_ref,
                   g_ref, be_ref, batch_ref, l1W_ref, l1b_ref, l2W_ref,
                   l2b_ref, out_ref):
    h = h_ref[...]
    agg = aggs_ref[0, :N, :] + aggs_ref[1, :N, :]
    z = (1.0 + eps_ref[...]) * h + agg
    a = jnp.maximum(
        jnp.dot(z, W1_ref[...], preferred_element_type=jnp.float32)
        + b1_ref[...], 0.0)
    b = jnp.maximum(
        jnp.dot(a, W2_ref[...], preferred_element_type=jnp.float32)
        + b2_ref[...], 0.0)
    mean = jnp.mean(b, axis=0)
    var = jnp.mean(b * b, axis=0) - mean * mean
    hn = (b - mean) * lax.rsqrt(var + 1e-5) * g_ref[...] + be_ref[...]
    bids = batch_ref[0, :]
    onehot = (lax.broadcasted_iota(jnp.int32, (G, N), 0)
              == bids[None, :]).astype(jnp.float32)
    sums = jnp.dot(onehot, hn, preferred_element_type=jnp.float32)
    cnt = jnp.sum(onehot, axis=1)
    pooled = sums / jnp.maximum(cnt, 1.0)[:, None]
    t = jnp.maximum(
        jnp.dot(pooled, l1W_ref[...], preferred_element_type=jnp.float32)
        + l1b_ref[...], 0.0)
    o = jnp.dot(t, l2W_ref[...], preferred_element_type=jnp.float32) \
        + l2b_ref[...]
    m = jnp.max(o, axis=1, keepdims=True)
    lse = jnp.log(jnp.sum(jnp.exp(o - m), axis=1, keepdims=True)) + m
    out_ref[...] = o - lse


@jax.jit
def _tc_final(h, aggs, epsb, W1, b1, W2, b2, g, be, batch2, l1W, l1b, l2W,
              l2b):
    return pl.pallas_call(
        _tc_final_body,
        out_shape=jax.ShapeDtypeStruct((G, OUT), jnp.float32),
    )(h, aggs, epsb, W1, b1, W2, b2, g, be, batch2, l1W, l1b, l2W, l2b)


def kernel(x, edge_index, batch,
           W1_0, b1_0, W2_0, b2_0, g_0, be_0, eps_0,
           W1_1, b1_1, W2_1, b2_1, g_1, be_1, eps_1,
           W1_2, b1_2, W2_2, b2_2, g_2, be_2, eps_2,
           lin1_W, lin1_b, lin2_W, lin2_b):
    batch2 = batch.reshape(1, N)

    params = [
        (W1_0, b1_0, W2_0, b2_0, g_0, be_0, eps_0),
        (W1_1, b1_1, W2_1, b2_1, g_1, be_1, eps_1),
        (W1_2, b1_2, W2_2, b2_2, g_2, be_2, eps_2),
    ]
    h = x
    for l in range(NUM_LAYERS):
        W1, b1, W2, b2, g, be, eps = params[l]
        aggs = _sc_agg(h, edge_index)
        epsb = jnp.broadcast_to(eps.reshape(1, 1), (1, H))
        b1r, b2r = b1.reshape(1, H), b2.reshape(1, H)
        gr, ber = g.reshape(1, H), be.reshape(1, H)
        if l < NUM_LAYERS - 1:
            h = _tc_layer(h, aggs, epsb, W1, b1r, W2, b2r, gr, ber)
        else:
            out = _tc_final(h, aggs, epsb, W1, b1r, W2, b2r, gr, ber,
                            batch2, lin1_W, lin1_b.reshape(1, H), lin2_W,
                            lin2_b.reshape(1, OUT))
    return out
